# baseline (speedup 1.0000x reference)
import jax
import jax.numpy as jnp
from jax import lax
from jax.experimental import pallas as pl
from jax.experimental.pallas import tpu as pltpu

_N = 4
_P = 256
_L = 50176
_CHUNK = 1024
_NCHUNK = _L // _CHUNK


def _interp_matrix(out_len: int, in_len: int) -> jnp.ndarray:
    ys = jnp.linspace(0.0, in_len - 1.0, out_len)
    y0 = jnp.floor(ys).astype(jnp.int32)
    y1 = jnp.minimum(y0 + 1, in_len - 1)
    wy = ys - y0.astype(ys.dtype)
    rows = jnp.arange(out_len)
    m = jnp.zeros((out_len, in_len), jnp.float32)
    m = m.at[rows, y0].add(1.0 - wy)
    m = m.at[rows, y1].add(wy)
    return m


def _body(x_ref, t_ref, m_ref, tf_ref, lo_ref, hi_ref, wy_ref, wxt_ref, out_ref):
    k_tot = 0.0
    sg_tot = 0.0
    sg2_tot = 0.0
    chx_tot = 0.0
    chy_tot = 0.0
    for b in range(_N):
        up = jnp.dot(wy_ref[...], x_ref[b], preferred_element_type=jnp.float32)
        up = jnp.dot(up, wxt_ref[...], preferred_element_type=jnp.float32)
        g = jnp.log(up) - jnp.log(t_ref[b])
        m = m_ref[b] > 0.0
        k_tot = k_tot + jnp.sum(m_ref[b])
        sg_tot = sg_tot + jnp.sum(jnp.where(m, g, 0.0))
        sg2_tot = sg2_tot + jnp.sum(jnp.where(m, g * g, 0.0))

        c = 0.5 * (lo_ref[b] + hi_ref[b])

        def chunk_body(ci, carry):
            dxy, syx, cnt = carry
            t = tf_ref[b, pl.ds(ci, 1), :]
            valid = t >= 0.001
            d = c - t
            d = d * d
            dxy = jnp.minimum(
                dxy, jnp.min(jnp.where(valid, d, 1e10), axis=1, keepdims=True))
            dmin = jnp.min(d, axis=0, keepdims=True)
            syx = syx + jnp.where(valid, dmin, 0.0)
            cnt = cnt + valid.astype(jnp.float32)
            return dxy, syx, cnt

        dxy0 = jnp.full((_P, 1), 1e10, jnp.float32)
        syx0 = jnp.zeros((1, _CHUNK), jnp.float32)
        cnt0 = jnp.zeros((1, _CHUNK), jnp.float32)
        dxy, syx, cnt = lax.fori_loop(0, _NCHUNK, chunk_body, (dxy0, syx0, cnt0))
        chx_tot = chx_tot + jnp.sum(dxy) / float(_P)
        chy_tot = chy_tot + jnp.sum(syx) / jnp.sum(cnt)

    mean_g = sg_tot / k_tot
    var_g = (sg2_tot - k_tot * mean_g * mean_g) / (k_tot - 1.0)
    loss1 = 10.0 * jnp.sqrt(var_g + 0.5 * mean_g * mean_g)
    loss2 = (chx_tot + chy_tot) / float(_N)
    out_ref[0, 0] = loss1 + 0.1 * loss2


@jax.jit
def kernel(bins, input, target, mask):
    n, _, h, w = input.shape
    H, W = target.shape[-2], target.shape[-1]
    wy = _interp_matrix(H, h)
    wxt = _interp_matrix(W, w).T
    x = input[:, 0]
    maskf = mask.astype(jnp.float32)
    tflat = target.reshape(n, _NCHUNK, _CHUNK)
    lo = bins[:, :-1][..., None]
    hi = bins[:, 1:][..., None]
    out = pl.pallas_call(
        _body,
        out_shape=jax.ShapeDtypeStruct((1, 1), jnp.float32),
        out_specs=pl.BlockSpec(memory_space=pltpu.SMEM),
    )(x, target, maskf, tflat, lo, hi, wy, wxt)
    return out[0, 0]
